# Initial kernel scaffold; baseline (speedup 1.0000x reference)
import jax
import jax.numpy as jnp
from jax.experimental import pallas as pl


def kernel(h, g, edge_idx, edge_type):
    raise NotImplementedError("write your pallas kernel here")


# speedup vs baseline: 5.4865x; 5.4865x over previous
import functools

import jax
import jax.numpy as jnp
from jax import lax
from jax.experimental import pallas as pl
from jax.experimental.pallas import tpu as pltpu
from jax.experimental.pallas import tpu_sc as plsc

_NC = 2
_NS = 16
_NW = _NC * _NS
_B = 80
_L = 16


@functools.partial(jax.jit, static_argnums=(5, 6, 7))
def _run(row, col, et, h, g, E, epw, nblk):
    D = h.shape[1]
    mesh = plsc.VectorSubcoreMesh(core_axis_name="c", subcore_axis_name="s")

    @functools.partial(
        pl.kernel,
        mesh=mesh,
        out_type=jax.ShapeDtypeStruct((E,), jnp.float32),
        compiler_params=pltpu.CompilerParams(needs_layout_passes=False),
        scratch_types=[
            pltpu.VMEM((nblk, _B), jnp.int32),
            pltpu.VMEM((nblk, _B), jnp.int32),
            pltpu.VMEM((nblk, _B), jnp.int32),
            pltpu.VMEM((_B, D), jnp.float32),
            pltpu.VMEM((_B, D), jnp.float32),
            pltpu.VMEM((_B, D), jnp.float32),
            pltpu.VMEM((epw,), jnp.float32),
            pltpu.SemaphoreType.DMA,
        ],
    )
    def k(rows_hbm, cols_hbm, ets_hbm, h_hbm, g_hbm, out_hbm,
          ridx, cidx, eidx, hr_v, hc_v, gr_v, out_v, sem):
        cid = lax.axis_index("c")
        sid = lax.axis_index("s")
        wid = sid * _NC + cid

        pltpu.sync_copy(rows_hbm.at[wid], ridx)
        pltpu.sync_copy(cols_hbm.at[wid], cidx)
        pltpu.sync_copy(ets_hbm.at[wid], eidx)

        lane = lax.iota(jnp.int32, _L)
        unroll = 8

        def block_body(b, carry):
            c1 = pltpu.async_copy(h_hbm.at[ridx.at[b]], hr_v, sem)
            c2 = pltpu.async_copy(h_hbm.at[cidx.at[b]], hc_v, sem)
            c3 = pltpu.async_copy(g_hbm.at[eidx.at[b]], gr_v, sem)
            c1.wait()
            c2.wait()
            c3.wait()

            def grp_body(grp, carry2):
                erange = grp * _L + lane

                def d_body(t, acc):
                    for k in range(unroll):
                        dv = (t * unroll + k + lane) & (D - 1)
                        a = plsc.load_gather(hr_v, [erange, dv])
                        r = plsc.load_gather(gr_v, [erange, dv])
                        c = plsc.load_gather(hc_v, [erange, dv])
                        acc = acc + jnp.abs(a + r - c)
                    return acc

                acc = lax.fori_loop(0, D // unroll, d_body,
                                    jnp.zeros((_L,), jnp.float32))
                out_v[pl.ds(b * _B + grp * _L, _L)] = acc
                return carry2

            return lax.fori_loop(0, _B // _L, grp_body, carry)

        lax.fori_loop(0, nblk, block_body, 0)
        pltpu.sync_copy(out_v, out_hbm.at[pl.ds(wid * epw, epw)])

    return k(row, col, et, h, g)


def kernel(h, g, edge_idx, edge_type):
    E = edge_type.shape[0]
    epw = E // _NW
    nblk = epw // _B
    row = edge_idx[0].astype(jnp.int32).reshape(_NW, nblk, _B)
    col = edge_idx[1].astype(jnp.int32).reshape(_NW, nblk, _B)
    et = edge_type.astype(jnp.int32).reshape(_NW, nblk, _B)
    return _run(row, col, et, h.astype(jnp.float32), g.astype(jnp.float32),
                E, epw, nblk)


# speedup vs baseline: 8.4714x; 1.5441x over previous
import functools

import jax
import jax.numpy as jnp
from jax import lax
from jax.experimental import pallas as pl
from jax.experimental.pallas import tpu as pltpu
from jax.experimental.pallas import tpu_sc as plsc

_NC = 2
_NS = 16
_NW = _NC * _NS
_B = 80
_L = 16


@functools.partial(jax.jit, static_argnums=(5, 6, 7))
def _run(row, col, et, h, g, E, epw, nblk):
    D = h.shape[1]
    mesh = plsc.VectorSubcoreMesh(core_axis_name="c", subcore_axis_name="s")

    @functools.partial(
        pl.kernel,
        mesh=mesh,
        out_type=jax.ShapeDtypeStruct((E,), jnp.float32),
        compiler_params=pltpu.CompilerParams(needs_layout_passes=False),
        scratch_types=[
            pltpu.VMEM((nblk, _B), jnp.int32),
            pltpu.VMEM((nblk, _B), jnp.int32),
            pltpu.VMEM((nblk, _B), jnp.int32),
            pltpu.VMEM((2, _B, D), jnp.float32),
            pltpu.VMEM((2, _B, D), jnp.float32),
            pltpu.VMEM((2, _B, D), jnp.float32),
            pltpu.VMEM((epw,), jnp.float32),
            pltpu.SemaphoreType.DMA,
            pltpu.SemaphoreType.DMA,
        ],
    )
    def k(rows_hbm, cols_hbm, ets_hbm, h_hbm, g_hbm, out_hbm,
          ridx, cidx, eidx, hr_v, hc_v, gr_v, out_v, sem0, sem1):
        cid = lax.axis_index("c")
        sid = lax.axis_index("s")
        wid = sid * _NC + cid

        pltpu.sync_copy(rows_hbm.at[wid], ridx)
        pltpu.sync_copy(cols_hbm.at[wid], cidx)
        pltpu.sync_copy(ets_hbm.at[wid], eidx)

        sems = (sem0, sem1)
        lane = lax.iota(jnp.int32, _L)
        unroll = 8

        def issue(b, j):
            pltpu.async_copy(h_hbm.at[ridx.at[b]], hr_v.at[j], sems[j])
            pltpu.async_copy(h_hbm.at[cidx.at[b]], hc_v.at[j], sems[j])
            pltpu.async_copy(g_hbm.at[eidx.at[b]], gr_v.at[j], sems[j])

        def drain(b, j):
            pltpu.make_async_copy(h_hbm.at[ridx.at[b]], hr_v.at[j],
                                  sems[j]).wait()
            pltpu.make_async_copy(h_hbm.at[cidx.at[b]], hc_v.at[j],
                                  sems[j]).wait()
            pltpu.make_async_copy(g_hbm.at[eidx.at[b]], gr_v.at[j],
                                  sems[j]).wait()

        def compute(b, j):
            hr, hc, gr = hr_v.at[j], hc_v.at[j], gr_v.at[j]

            def grp_body(grp, carry2):
                erange = grp * _L + lane

                def d_body(t, acc):
                    for kk in range(unroll):
                        dv = (t * unroll + kk + lane) & (D - 1)
                        a = plsc.load_gather(hr, [erange, dv])
                        r = plsc.load_gather(gr, [erange, dv])
                        c = plsc.load_gather(hc, [erange, dv])
                        acc = acc + jnp.abs(a + r - c)
                    return acc

                acc = lax.fori_loop(0, D // unroll, d_body,
                                    jnp.zeros((_L,), jnp.float32))
                out_v[pl.ds(b * _B + grp * _L, _L)] = acc
                return carry2

            lax.fori_loop(0, _B // _L, grp_body, 0)

        issue(0, 0)
        issue(1, 1)

        def pair_body(i, carry):
            for j in range(2):
                b = 2 * i + j
                drain(b, j)
                compute(b, j)

                @pl.when(b + 2 < nblk)
                def _():
                    issue(b + 2, j)
            return carry

        lax.fori_loop(0, nblk // 2, pair_body, 0)
        if nblk % 2:
            drain(nblk - 1, 0)
            compute(nblk - 1, 0)

        pltpu.sync_copy(out_v, out_hbm.at[pl.ds(wid * epw, epw)])

    return k(row, col, et, h, g)


def kernel(h, g, edge_idx, edge_type):
    E = edge_type.shape[0]
    epw = E // _NW
    nblk = epw // _B
    row = edge_idx[0].astype(jnp.int32).reshape(_NW, nblk, _B)
    col = edge_idx[1].astype(jnp.int32).reshape(_NW, nblk, _B)
    et = edge_type.astype(jnp.int32).reshape(_NW, nblk, _B)
    return _run(row, col, et, h.astype(jnp.float32), g.astype(jnp.float32),
                E, epw, nblk)


# speedup vs baseline: 11.1070x; 1.3111x over previous
import functools

import jax
import jax.numpy as jnp
from jax import lax
from jax.experimental import pallas as pl
from jax.experimental.pallas import tpu as pltpu
from jax.experimental.pallas import tpu_sc as plsc

_NC = 2
_NS = 16
_NW = _NC * _NS
_B = 80
_L = 16


@functools.partial(jax.jit, static_argnums=(5, 6, 7))
def _run(row, col, et, h, gpk, E, epw, nblk):
    D = h.shape[1]
    W = D // 2
    GW = gpk.shape[0]
    mesh = plsc.VectorSubcoreMesh(core_axis_name="c", subcore_axis_name="s")

    @functools.partial(
        pl.kernel,
        mesh=mesh,
        out_type=jax.ShapeDtypeStruct((E,), jnp.float32),
        compiler_params=pltpu.CompilerParams(needs_layout_passes=False),
        scratch_types=[
            pltpu.VMEM((epw,), jnp.int32),
            pltpu.VMEM((epw,), jnp.int32),
            pltpu.VMEM((epw,), jnp.int32),
            pltpu.VMEM((2, _B, D), jnp.float32),
            pltpu.VMEM((2, _B, D), jnp.float32),
            pltpu.VMEM((GW,), jnp.int32),
            pltpu.VMEM((epw,), jnp.float32),
            pltpu.SemaphoreType.DMA,
            pltpu.SemaphoreType.DMA,
        ],
    )
    def k(rows_hbm, cols_hbm, ets_hbm, h_hbm, gpk_hbm, out_hbm,
          ridx, cidx, eidx, hr_v, hc_v, gpk_v, out_v, sem0, sem1):
        cid = lax.axis_index("c")
        sid = lax.axis_index("s")
        wid = sid * _NC + cid

        pltpu.sync_copy(rows_hbm.at[wid], ridx)
        pltpu.sync_copy(cols_hbm.at[wid], cidx)
        pltpu.sync_copy(ets_hbm.at[wid], eidx)
        pltpu.sync_copy(gpk_hbm, gpk_v)

        sems = (sem0, sem1)
        lane = lax.iota(jnp.int32, _L)
        unroll = 4
        himask = jnp.full((_L,), -65536, jnp.int32)

        def issue(b, j):
            pltpu.async_copy(h_hbm.at[ridx.at[pl.ds(b * _B, _B)]],
                             hr_v.at[j], sems[j])
            pltpu.async_copy(h_hbm.at[cidx.at[pl.ds(b * _B, _B)]],
                             hc_v.at[j], sems[j])

        def drain(b, j):
            pltpu.make_async_copy(h_hbm.at[ridx.at[pl.ds(b * _B, _B)]],
                                  hr_v.at[j], sems[j]).wait()
            pltpu.make_async_copy(h_hbm.at[cidx.at[pl.ds(b * _B, _B)]],
                                  hc_v.at[j], sems[j]).wait()

        def compute(b, j):
            hr, hc = hr_v.at[j], hc_v.at[j]

            def grp_body(grp, carry2):
                erange = grp * _L + lane
                etv = eidx[pl.ds(b * _B + grp * _L, _L)]
                gbase = etv << 6 if W == 64 else etv * W

                def d_body(t, accs):
                    acc_e, acc_o = accs
                    for kk in range(unroll):
                        wv = (t * unroll + kk + lane) & (W - 1)
                        de = wv << 1
                        do = de | 1
                        gw = plsc.load_gather(gpk_v, [gbase + wv])
                        ge = plsc.bitcast(gw << 16, jnp.float32)
                        go = plsc.bitcast(gw & himask, jnp.float32)
                        a0 = plsc.load_gather(hr, [erange, de])
                        c0 = plsc.load_gather(hc, [erange, de])
                        a1 = plsc.load_gather(hr, [erange, do])
                        c1 = plsc.load_gather(hc, [erange, do])
                        acc_e = acc_e + jnp.abs(a0 + ge - c0)
                        acc_o = acc_o + jnp.abs(a1 + go - c1)
                    return acc_e, acc_o

                z = jnp.zeros((_L,), jnp.float32)
                acc_e, acc_o = lax.fori_loop(0, W // unroll, d_body, (z, z))
                out_v[pl.ds(b * _B + grp * _L, _L)] = acc_e + acc_o
                return carry2

            lax.fori_loop(0, _B // _L, grp_body, 0)

        issue(0, 0)
        issue(1, 1)

        def pair_body(i, carry):
            for j in range(2):
                b = 2 * i + j
                drain(b, j)
                compute(b, j)

                @pl.when(b + 2 < nblk)
                def _():
                    issue(b + 2, j)

            return carry

        lax.fori_loop(0, nblk // 2, pair_body, 0)
        if nblk % 2:
            drain(nblk - 1, 0)
            compute(nblk - 1, 0)

        pltpu.sync_copy(out_v, out_hbm.at[pl.ds(wid * epw, epw)])

    return k(row, col, et, h, gpk)


def kernel(h, g, edge_idx, edge_type):
    E = edge_type.shape[0]
    epw = E // _NW
    nblk = epw // _B
    row = edge_idx[0].astype(jnp.int32).reshape(_NW, epw)
    col = edge_idx[1].astype(jnp.int32).reshape(_NW, epw)
    et = edge_type.astype(jnp.int32).reshape(_NW, epw)
    g_bf = g.astype(jnp.float32).astype(jnp.bfloat16)
    gpk = lax.bitcast_convert_type(
        g_bf.reshape(g.shape[0], g.shape[1] // 2, 2), jnp.int32).reshape(-1)
    return _run(row, col, et, h.astype(jnp.float32), gpk, E, epw, nblk)


# speedup vs baseline: 12.5153x; 1.1268x over previous
import functools

import jax
import jax.numpy as jnp
from jax import lax
from jax.experimental import pallas as pl
from jax.experimental.pallas import tpu as pltpu
from jax.experimental.pallas import tpu_sc as plsc

_NC = 2
_NS = 16
_NW = _NC * _NS
_B = 80
_L = 16


@functools.partial(jax.jit, static_argnums=(5, 6, 7))
def _run(row, col, et, hpk, gpk, E, epw, nblk):
    W = hpk.shape[1]
    GW = gpk.shape[0]
    mesh = plsc.VectorSubcoreMesh(core_axis_name="c", subcore_axis_name="s")

    @functools.partial(
        pl.kernel,
        mesh=mesh,
        out_type=jax.ShapeDtypeStruct((E,), jnp.float32),
        compiler_params=pltpu.CompilerParams(needs_layout_passes=False,
                                             use_tc_tiling_on_sc=False),
        scratch_types=[
            pltpu.VMEM((epw,), jnp.int32),
            pltpu.VMEM((epw,), jnp.int32),
            pltpu.VMEM((epw,), jnp.int32),
            pltpu.VMEM((2, _B, W), jnp.int32),
            pltpu.VMEM((2, _B, W), jnp.int32),
            pltpu.VMEM((GW,), jnp.int32),
            pltpu.VMEM((epw,), jnp.float32),
            pltpu.SemaphoreType.DMA,
            pltpu.SemaphoreType.DMA,
        ],
    )
    def k(rows_hbm, cols_hbm, ets_hbm, hpk_hbm, gpk_hbm, out_hbm,
          ridx, cidx, eidx, hr_v, hc_v, gpk_v, out_v, sem0, sem1):
        cid = lax.axis_index("c")
        sid = lax.axis_index("s")
        wid = sid * _NC + cid

        pltpu.sync_copy(rows_hbm.at[wid], ridx)
        pltpu.sync_copy(cols_hbm.at[wid], cidx)
        pltpu.sync_copy(ets_hbm.at[wid], eidx)
        pltpu.sync_copy(gpk_hbm, gpk_v)

        sems = (sem0, sem1)
        lane = lax.iota(jnp.int32, _L)
        unroll = 8

        def issue(b, j):
            pltpu.async_copy(hpk_hbm.at[ridx.at[pl.ds(b * _B, _B)]],
                             hr_v.at[j], sems[j])
            pltpu.async_copy(hpk_hbm.at[cidx.at[pl.ds(b * _B, _B)]],
                             hc_v.at[j], sems[j])

        def drain(b, j):
            pltpu.make_async_copy(hpk_hbm.at[ridx.at[pl.ds(b * _B, _B)]],
                                  hr_v.at[j], sems[j]).wait()
            pltpu.make_async_copy(hpk_hbm.at[cidx.at[pl.ds(b * _B, _B)]],
                                  hc_v.at[j], sems[j]).wait()

        def compute(b, j):
            hr, hc = hr_v.at[j], hc_v.at[j]

            def grp_body(grp, carry2):
                erange = grp * _L + lane
                etv = eidx[pl.ds(b * _B + grp * _L, _L)]
                gbase = etv << 6 if W == 64 else etv * W

                def d_body(t, accs):
                    acc_e, acc_o = accs
                    for kk in range(unroll):
                        wv = (t * unroll + kk + lane) & (W - 1)
                        gw = plsc.load_gather(gpk_v, [gbase + wv])
                        aw = plsc.load_gather(hr, [erange, wv])
                        cw = plsc.load_gather(hc, [erange, wv])
                        s = (plsc.bitcast(aw, jnp.bfloat16)
                             + plsc.bitcast(gw, jnp.bfloat16)
                             - plsc.bitcast(cw, jnp.bfloat16))
                        s = jnp.abs(s)
                        se, so = plsc.unpack(
                            s, format=plsc.PackFormat.INTERLEAVED,
                            preferred_element_type=jnp.float32)
                        acc_e = acc_e + se
                        acc_o = acc_o + so
                    return acc_e, acc_o

                z = jnp.zeros((_L,), jnp.float32)
                acc_e, acc_o = lax.fori_loop(0, W // unroll, d_body, (z, z))
                out_v[pl.ds(b * _B + grp * _L, _L)] = acc_e + acc_o
                return carry2

            lax.fori_loop(0, _B // _L, grp_body, 0)

        issue(0, 0)
        issue(1, 1)

        def pair_body(i, carry):
            for j in range(2):
                b = 2 * i + j
                drain(b, j)
                compute(b, j)

                @pl.when(b + 2 < nblk)
                def _():
                    issue(b + 2, j)

            return carry

        lax.fori_loop(0, nblk // 2, pair_body, 0)
        if nblk % 2:
            drain(nblk - 1, 0)
            compute(nblk - 1, 0)

        pltpu.sync_copy(out_v, out_hbm.at[pl.ds(wid * epw, epw)])

    return k(row, col, et, hpk, gpk)


def kernel(h, g, edge_idx, edge_type):
    E = edge_type.shape[0]
    epw = E // _NW
    nblk = epw // _B
    row = edge_idx[0].astype(jnp.int32).reshape(_NW, epw)
    col = edge_idx[1].astype(jnp.int32).reshape(_NW, epw)
    et = edge_type.astype(jnp.int32).reshape(_NW, epw)
    g_bf = g.astype(jnp.float32).astype(jnp.bfloat16)
    gpk = lax.bitcast_convert_type(
        g_bf.reshape(g.shape[0], g.shape[1] // 2, 2), jnp.int32).reshape(-1)
    h_bf = h.astype(jnp.float32).astype(jnp.bfloat16)
    hpk = lax.bitcast_convert_type(
        h_bf.reshape(h.shape[0], h.shape[1] // 2, 2), jnp.int32)
    return _run(row, col, et, hpk, gpk, E, epw, nblk)


# speedup vs baseline: 13.6638x; 1.0918x over previous
import functools

import jax
import jax.numpy as jnp
from jax import lax
from jax.experimental import pallas as pl
from jax.experimental.pallas import tpu as pltpu
from jax.experimental.pallas import tpu_sc as plsc

_NC = 2
_NS = 16
_NW = _NC * _NS
_B = 80
_L = 16


@functools.partial(jax.jit, static_argnums=(5, 6, 7))
def _run(row, col, et, hpk, gpk, E, epw, nblk):
    W = hpk.shape[1]
    GW = gpk.shape[0]
    mesh = plsc.VectorSubcoreMesh(core_axis_name="c", subcore_axis_name="s")

    @functools.partial(
        pl.kernel,
        mesh=mesh,
        out_type=jax.ShapeDtypeStruct((E,), jnp.float32),
        compiler_params=pltpu.CompilerParams(needs_layout_passes=False,
                                             use_tc_tiling_on_sc=False),
        scratch_types=[
            pltpu.VMEM((epw,), jnp.int32),
            pltpu.VMEM((epw,), jnp.int32),
            pltpu.VMEM((epw,), jnp.int32),
            pltpu.VMEM((4, _B, W), jnp.int32),
            pltpu.VMEM((4, _B, W), jnp.int32),
            pltpu.VMEM((GW,), jnp.int32),
            pltpu.VMEM((epw,), jnp.float32),
            pltpu.SemaphoreType.DMA,
            pltpu.SemaphoreType.DMA,
            pltpu.SemaphoreType.DMA,
            pltpu.SemaphoreType.DMA,
        ],
    )
    def k(rows_hbm, cols_hbm, ets_hbm, hpk_hbm, gpk_hbm, out_hbm,
          ridx, cidx, eidx, hr_v, hc_v, gpk_v, out_v,
          sem0, sem1, sem2, sem3):
        cid = lax.axis_index("c")
        sid = lax.axis_index("s")
        wid = sid * _NC + cid

        pltpu.sync_copy(rows_hbm.at[wid], ridx)
        pltpu.sync_copy(cols_hbm.at[wid], cidx)
        pltpu.sync_copy(ets_hbm.at[wid], eidx)
        pltpu.sync_copy(gpk_hbm, gpk_v)

        sems = (sem0, sem1, sem2, sem3)
        nslot = 4
        lane = lax.iota(jnp.int32, _L)
        unroll = 8

        def issue(b, j):
            pltpu.async_copy(hpk_hbm.at[ridx.at[pl.ds(b * _B, _B)]],
                             hr_v.at[j], sems[j])
            pltpu.async_copy(hpk_hbm.at[cidx.at[pl.ds(b * _B, _B)]],
                             hc_v.at[j], sems[j])

        def drain(b, j):
            pltpu.make_async_copy(hpk_hbm.at[ridx.at[pl.ds(b * _B, _B)]],
                                  hr_v.at[j], sems[j]).wait()
            pltpu.make_async_copy(hpk_hbm.at[cidx.at[pl.ds(b * _B, _B)]],
                                  hc_v.at[j], sems[j]).wait()

        def compute(b, j):
            hr, hc = hr_v.at[j], hc_v.at[j]

            def grp_body(grp, carry2):
                erange = grp * _L + lane
                etv = eidx[pl.ds(b * _B + grp * _L, _L)]
                gbase = etv << 6 if W == 64 else etv * W

                def d_body(t, accs):
                    acc_e, acc_o = accs
                    for kk in range(unroll):
                        wv = (t * unroll + kk + lane) & (W - 1)
                        gw = plsc.load_gather(gpk_v, [gbase + wv])
                        aw = plsc.load_gather(hr, [erange, wv])
                        cw = plsc.load_gather(hc, [erange, wv])
                        s = (plsc.bitcast(aw, jnp.bfloat16)
                             + plsc.bitcast(gw, jnp.bfloat16)
                             - plsc.bitcast(cw, jnp.bfloat16))
                        s = jnp.abs(s)
                        se, so = plsc.unpack(
                            s, format=plsc.PackFormat.INTERLEAVED,
                            preferred_element_type=jnp.float32)
                        acc_e = acc_e + se
                        acc_o = acc_o + so
                    return acc_e, acc_o

                z = jnp.zeros((_L,), jnp.float32)
                acc_e, acc_o = lax.fori_loop(0, W // unroll, d_body, (z, z))
                out_v[pl.ds(b * _B + grp * _L, _L)] = acc_e + acc_o
                return carry2

            lax.fori_loop(0, _B // _L, grp_body, 0)

        for j in range(nslot):
            issue(j, j)

        def ring_body(i, carry):
            for j in range(nslot):
                b = i * nslot + j
                drain(b, j)
                compute(b, j)

                @pl.when(b + nslot < nblk)
                def _():
                    issue(b + nslot, j)

            return carry

        lax.fori_loop(0, nblk // nslot, ring_body, 0)
        for r in range(nblk % nslot):
            b = (nblk // nslot) * nslot + r
            drain(b, b % nslot)
            compute(b, b % nslot)

        pltpu.sync_copy(out_v, out_hbm.at[pl.ds(wid * epw, epw)])

    return k(row, col, et, hpk, gpk)


def kernel(h, g, edge_idx, edge_type):
    E = edge_type.shape[0]
    epw = E // _NW
    nblk = epw // _B
    row = edge_idx[0].astype(jnp.int32).reshape(_NW, epw)
    col = edge_idx[1].astype(jnp.int32).reshape(_NW, epw)
    et = edge_type.astype(jnp.int32).reshape(_NW, epw)
    g_bf = g.astype(jnp.float32).astype(jnp.bfloat16)
    gpk = lax.bitcast_convert_type(
        g_bf.reshape(g.shape[0], g.shape[1] // 2, 2), jnp.int32).reshape(-1)
    h_bf = h.astype(jnp.float32).astype(jnp.bfloat16)
    hpk = lax.bitcast_convert_type(
        h_bf.reshape(h.shape[0], h.shape[1] // 2, 2), jnp.int32)
    return _run(row, col, et, hpk, gpk, E, epw, nblk)


# speedup vs baseline: 16.9258x; 1.2387x over previous
import functools

import jax
import jax.numpy as jnp
from jax import lax
from jax.experimental import pallas as pl
from jax.experimental.pallas import tpu as pltpu
from jax.experimental.pallas import tpu_sc as plsc

_NC = 2
_NS = 16
_NW = _NC * _NS
_B = 80
_L = 16


@functools.partial(jax.jit, static_argnums=(4, 5, 6))
def _run(eidx2, et, hpk, gpk, E, epw, nblk):
    W = hpk.shape[1]
    GW = gpk.shape[0]
    mesh = plsc.VectorSubcoreMesh(core_axis_name="c", subcore_axis_name="s")

    @functools.partial(
        pl.kernel,
        mesh=mesh,
        out_type=jax.ShapeDtypeStruct((E,), jnp.float32),
        compiler_params=pltpu.CompilerParams(needs_layout_passes=False,
                                             use_tc_tiling_on_sc=False),
        scratch_types=[
            pltpu.VMEM((epw,), jnp.int32),
            pltpu.VMEM((epw,), jnp.int32),
            pltpu.VMEM((epw,), jnp.int32),
            pltpu.VMEM((4, _B, W), jnp.int32),
            pltpu.VMEM((4, _B, W), jnp.int32),
            pltpu.VMEM((GW,), jnp.int32),
            pltpu.VMEM((epw,), jnp.float32),
            pltpu.SemaphoreType.DMA,
            pltpu.SemaphoreType.DMA,
            pltpu.SemaphoreType.DMA,
            pltpu.SemaphoreType.DMA,
        ],
    )
    def k(eidx2_hbm, et_hbm, hpk_hbm, gpk_hbm, out_hbm,
          ridx, cidx, eidx, hr_v, hc_v, gpk_v, out_v,
          sem0, sem1, sem2, sem3):
        cid = lax.axis_index("c")
        sid = lax.axis_index("s")
        wid = sid * _NC + cid

        pltpu.sync_copy(eidx2_hbm.at[0, pl.ds(wid * epw, epw)], ridx)
        pltpu.sync_copy(eidx2_hbm.at[1, pl.ds(wid * epw, epw)], cidx)
        pltpu.sync_copy(et_hbm.at[pl.ds(wid * epw, epw)], eidx)
        pltpu.sync_copy(gpk_hbm, gpk_v)

        sems = (sem0, sem1, sem2, sem3)
        nslot = 4
        lane = lax.iota(jnp.int32, _L)
        unroll = 8

        def issue(b, j):
            pltpu.async_copy(hpk_hbm.at[ridx.at[pl.ds(b * _B, _B)]],
                             hr_v.at[j], sems[j])
            pltpu.async_copy(hpk_hbm.at[cidx.at[pl.ds(b * _B, _B)]],
                             hc_v.at[j], sems[j])

        def drain(b, j):
            pltpu.make_async_copy(hpk_hbm.at[ridx.at[pl.ds(b * _B, _B)]],
                                  hr_v.at[j], sems[j]).wait()
            pltpu.make_async_copy(hpk_hbm.at[cidx.at[pl.ds(b * _B, _B)]],
                                  hc_v.at[j], sems[j]).wait()

        def compute(b, j):
            hr, hc = hr_v.at[j], hc_v.at[j]

            def grp_body(grp, carry2):
                erange = grp * _L + lane
                etv = eidx[pl.ds(b * _B + grp * _L, _L)]
                gbase = etv << 6 if W == 64 else etv * W

                def d_body(t, accs):
                    acc_e, acc_o = accs
                    for kk in range(unroll):
                        wv = (t * unroll + kk + lane) & (W - 1)
                        gw = plsc.load_gather(gpk_v, [gbase + wv])
                        aw = plsc.load_gather(hr, [erange, wv])
                        cw = plsc.load_gather(hc, [erange, wv])
                        s = (plsc.bitcast(aw, jnp.bfloat16)
                             + plsc.bitcast(gw, jnp.bfloat16)
                             - plsc.bitcast(cw, jnp.bfloat16))
                        s = jnp.abs(s)
                        se, so = plsc.unpack(
                            s, format=plsc.PackFormat.INTERLEAVED,
                            preferred_element_type=jnp.float32)
                        acc_e = acc_e + se
                        acc_o = acc_o + so
                    return acc_e, acc_o

                z = jnp.zeros((_L,), jnp.float32)
                acc_e, acc_o = lax.fori_loop(0, W // unroll, d_body, (z, z))
                out_v[pl.ds(b * _B + grp * _L, _L)] = acc_e + acc_o
                return carry2

            lax.fori_loop(0, _B // _L, grp_body, 0)

        for j in range(nslot):
            issue(j, j)

        def ring_body(i, carry):
            for j in range(nslot):
                b = i * nslot + j
                drain(b, j)
                compute(b, j)

                @pl.when(b + nslot < nblk)
                def _():
                    issue(b + nslot, j)

            return carry

        lax.fori_loop(0, nblk // nslot, ring_body, 0)
        for r in range(nblk % nslot):
            b = (nblk // nslot) * nslot + r
            drain(b, b % nslot)
            compute(b, b % nslot)

        pltpu.sync_copy(out_v, out_hbm.at[pl.ds(wid * epw, epw)])

    return k(eidx2, et, hpk, gpk)


def _pack_bf16_pairs(x):
    half = x.shape[1] // 2
    u = lax.bitcast_convert_type(x.astype(jnp.float32), jnp.uint32)
    rnd = u + jnp.uint32(0x7FFF) + ((u >> 16) & jnp.uint32(1))
    lo = rnd[:, :half] >> 16
    hi = rnd[:, half:] & jnp.uint32(0xFFFF0000)
    return lax.bitcast_convert_type(lo | hi, jnp.int32)


def kernel(h, g, edge_idx, edge_type):
    E = edge_type.shape[0]
    epw = E // _NW
    nblk = epw // _B
    hpk = _pack_bf16_pairs(h)
    gpk = _pack_bf16_pairs(g).reshape(-1)
    return _run(edge_idx.astype(jnp.int32), edge_type.astype(jnp.int32),
                hpk, gpk, E, epw, nblk)
